# baseline (speedup 1.0000x reference)
import functools

import jax
import jax.numpy as jnp
from jax import lax
from jax.experimental import pallas as pl
from jax.experimental.pallas import tpu as pltpu
from jax.experimental.pallas import tpu_sc as plsc

_B, _L, _V, _E, _A = 1024, 200, 100000, 128, 8
_SPLIT = 128
_REM = _L - _SPLIT
_OFFS = tuple(range(0, _L - 16, 16)) + (_L - 16,)
_NBUF = 4


def kernel(sequence, attrs_idxs, token_table, pos_table, attr_table0,
           attr_table1):
    seq_flat = sequence.reshape(_B * _L)
    a0 = attrs_idxs[0].reshape(_B * _L)
    a1 = attrs_idxs[1].reshape(_B * _L)

    info = plsc.get_sparse_core_info()
    nc, ns = info.num_cores, info.num_subcores
    nw = nc * ns
    rows_per_w = _B // nw

    mesh = plsc.VectorSubcoreMesh(core_axis_name="c", subcore_axis_name="s")

    buf_scratch = []
    for _ in range(_NBUF):
        buf_scratch += [
            pltpu.VMEM((_L,), jnp.int32),
            pltpu.VMEM((_L,), jnp.int32),
            pltpu.VMEM((_L,), jnp.int32),
            pltpu.VMEM((_L + 16,), jnp.int32),
            pltpu.VMEM((_L, _E), jnp.float32),
            pltpu.SemaphoreType.DMA,
            pltpu.SemaphoreType.DMA,
            pltpu.SemaphoreType.DMA,
            pltpu.SemaphoreType.DMA,
            pltpu.SemaphoreType.DMA,
        ]

    @functools.partial(
        pl.kernel,
        mesh=mesh,
        out_type=jax.ShapeDtypeStruct((_B, _L, _E), jnp.float32),
        scratch_types=[
            pltpu.VMEM((_A, _E), jnp.float32),
            pltpu.VMEM((_A, _E), jnp.float32),
            pltpu.VMEM((_A * _A, _E), jnp.float32),
            pltpu.VMEM_SHARED((_A * _A, _E), jnp.float32),
            pltpu.VMEM_SHARED((_L, _E), jnp.float32),
            pltpu.VMEM((_L,), jnp.int32),
        ] + buf_scratch,
    )
    def k(seq_hbm, a0_hbm, a1_hbm, token_hbm, pos_hbm, attr0_hbm, attr1_hbm,
          out_hbm, attr0_v, attr1_v, pair_v, pair_sh, pos_sh, posidx,
          *flat_bufs):
        cid = lax.axis_index("c")
        sid = lax.axis_index("s")
        wid = sid * nc + cid
        b0 = wid * rows_per_w

        bufs = tuple(flat_bufs[i * 10:(i + 1) * 10] for i in range(_NBUF))

        @pl.when(sid == 0)
        def _():
            pltpu.sync_copy(attr0_hbm, attr0_v)
            pltpu.sync_copy(attr1_hbm, attr1_v)

            def build_pair(i, carry):
                for j in range(_A):
                    for cb in range(_E // 16):
                        s = pl.ds(cb * 16, 16)
                        pair_v[i * _A + j, s] = attr0_v[i, s] + attr1_v[j, s]
                return carry

            lax.fori_loop(0, _A, build_pair, 0)
            pltpu.sync_copy(pair_v, pair_sh)
            pltpu.sync_copy(pos_hbm, pos_sh)

        iota = lax.iota(jnp.int32, 16)
        for off in _OFFS:
            posidx[pl.ds(off, 16)] = iota + off

        plsc.subcore_barrier()

        def idx_copies(c, buf):
            seqx, a0b, a1b, sem_i = buf[0], buf[1], buf[2], buf[7]
            base = (b0 + c) * _L
            return (
                pltpu.make_async_copy(seq_hbm.at[pl.ds(base, _SPLIT)],
                                      seqx.at[pl.ds(0, _SPLIT)], sem_i),
                pltpu.make_async_copy(
                    seq_hbm.at[pl.ds(base + _SPLIT, _REM)],
                    seqx.at[pl.ds(_SPLIT, _REM)], sem_i),
                pltpu.make_async_copy(a0_hbm.at[pl.ds(base, _L)], a0b,
                                      sem_i),
                pltpu.make_async_copy(a1_hbm.at[pl.ds(base, _L)], a1b,
                                      sem_i),
            )

        def compute_pidx(buf):
            a0b, a1b, pidx = buf[1], buf[2], buf[3]
            for off in _OFFS:
                s = pl.ds(off, 16)
                pidx[s] = a0b[s] * _A + a1b[s]

        def gather_copies(buf):
            seqx, rows, sem_g = buf[0], buf[4], buf[5]
            cp1 = pltpu.make_async_copy(
                token_hbm.at[seqx.at[pl.ds(0, _SPLIT)]],
                rows.at[pl.ds(0, _SPLIT)], sem_g)
            cp2 = pltpu.make_async_copy(
                token_hbm.at[seqx.at[pl.ds(_SPLIT, _REM)]],
                rows.at[pl.ds(_SPLIT, _REM)], sem_g)
            return cp1, cp2

        def wb_copies(c, buf):
            rows, sem_w = buf[4], buf[6]
            return (
                pltpu.make_async_copy(rows, out_hbm.at[b0 + c], sem_w),
            )

        def add_copies(buf):
            pidx, rows = buf[3], buf[4]
            sem_a = buf[8]
            return (
                pltpu.make_async_copy(
                    pair_sh.at[pidx.at[pl.ds(0, _SPLIT)]],
                    rows.at[pl.ds(0, _SPLIT)], sem_a),
                pltpu.make_async_copy(
                    pair_sh.at[pidx.at[pl.ds(_SPLIT, _REM)]],
                    rows.at[pl.ds(_SPLIT, _REM)], sem_a),
                pltpu.make_async_copy(
                    pos_sh.at[posidx.at[pl.ds(0, _SPLIT)]],
                    rows.at[pl.ds(0, _SPLIT)], sem_a),
                pltpu.make_async_copy(
                    pos_sh.at[posidx.at[pl.ds(_SPLIT, _REM)]],
                    rows.at[pl.ds(_SPLIT, _REM)], sem_a),
            )

        def step(c, prv, cur, nxt, nxt2):
            @pl.when(c < rows_per_w - 2)
            def _():
                for cp in idx_copies(c + 2, nxt2):
                    cp.start()

            @pl.when(c > 0)
            def _():
                for cp in add_copies(prv):
                    cp.wait()
                for cp in wb_copies(c - 1, prv):
                    cp.start()

            @pl.when(c >= _NBUF - 1)
            def _():
                for cp in wb_copies(c - (_NBUF - 1), nxt):
                    cp.wait()

            @pl.when(c < rows_per_w - 1)
            def _():
                for cp in idx_copies(c + 1, nxt):
                    cp.wait()
                for cp in gather_copies(nxt):
                    cp.start()
                compute_pidx(nxt)

            for cp in gather_copies(cur):
                cp.wait()

            for cp in add_copies(cur):
                cp.start(add=True)

        for cp in idx_copies(0, bufs[0]):
            cp.start()
        for cp in idx_copies(1, bufs[1]):
            cp.start()
        for cp in idx_copies(0, bufs[0]):
            cp.wait()
        for cp in gather_copies(bufs[0]):
            cp.start()
        compute_pidx(bufs[0])

        def quad_of_chunks(i, carry):
            c = _NBUF * i
            for ph in range(_NBUF):
                step(c + ph, bufs[(ph - 1) % _NBUF], bufs[ph],
                     bufs[(ph + 1) % _NBUF], bufs[(ph + 2) % _NBUF])
            return carry

        lax.fori_loop(0, rows_per_w // _NBUF, quad_of_chunks, 0)

        c_last = rows_per_w - 1
        for cp in add_copies(bufs[c_last % _NBUF]):
            cp.wait()
        for cp in wb_copies(c_last, bufs[c_last % _NBUF]):
            cp.start()
        for cm1 in (c_last - 2, c_last - 1, c_last):
            for cp in wb_copies(cm1, bufs[cm1 % _NBUF]):
                cp.wait()

    return k(seq_flat, a0, a1, token_table, pos_table, attr_table0,
             attr_table1)
